# baseline (device time: 147326 ns/iter reference)
import os

import jax
import jax.numpy as jnp
from jax import lax
from jax.experimental import pallas as pl
from jax.experimental.pallas import tpu as pltpu

_NO_COMM = bool(int(os.environ.get("A2A_NO_COMM", "0")))
_NO_EPI = bool(int(os.environ.get("A2A_NO_EPI", "0")))

N_DEV = 4
M_PER = 1024
K_BLK = 1024
N_TOTAL = 8192
N_CHUNK = 1024
NB = N_TOTAL // N_CHUNK


def kernel(x, w_mat):
    def body(x_ref, w_ref, out_ref, xstage_ref, xsend_ref, xg_ref, wbuf_ref,
             amax_ref, send_sems, recv_sems, a_send_sems, a_recv_sems,
             w_sems, x_sems):
        my = lax.axis_index("i")

        k_seq = [
            (None, my * K_BLK),
            (0, ((my - 1) % N_DEV) * K_BLK),
            (2, ((my + 1) % N_DEV) * K_BLK),
            (1, ((my + 2) % N_DEV) * K_BLK),
        ]
        flat = [(ki, nb) for ki in range(N_DEV) for nb in range(NB)]
        NSLOT = 2
        w_descs = {}

        def start_w(step):
            ki, nb = flat[step]
            _, krow = k_seq[ki]
            d = pltpu.make_async_copy(
                w_ref.at[pl.ds(krow, K_BLK), pl.ds(nb * N_CHUNK, N_CHUNK)],
                wbuf_ref.at[step % NSLOT],
                w_sems.at[step % NSLOT],
            )
            d.start()
            w_descs[step] = d

        def start_x(block_j):
            d = pltpu.make_async_copy(
                x_ref.at[pl.ds(block_j * M_PER, M_PER), :],
                xstage_ref,
                x_sems,
            )
            d.start()
            return d

        xdl = start_x(my)
        start_w(0)
        start_w(1)

        if not _NO_COMM:
            barrier = pltpu.get_barrier_semaphore()
            for dj in range(1, N_DEV):
                pl.semaphore_signal(
                    barrier, inc=1,
                    device_id=((my + dj) % N_DEV,),
                    device_id_type=pl.DeviceIdType.MESH,
                )
            pl.semaphore_wait(barrier, N_DEV - 1)

        x_sends = []

        def send_block(dj):
            j = (my + dj) % N_DEV
            rdma = pltpu.make_async_remote_copy(
                src_ref=xsend_ref.at[dj - 1],
                dst_ref=xg_ref.at[dj - 1],
                send_sem=send_sems.at[dj - 1],
                recv_sem=recv_sems.at[dj - 1],
                device_id=(j,),
                device_id_type=pl.DeviceIdType.MESH,
            )
            rdma.start()
            x_sends.append(rdma)

        xdl.wait()
        xsend_ref[N_DEV - 1] = xstage_ref[...].astype(jnp.bfloat16)
        x_loads = [start_x((my + 1) % N_DEV), None, None]

        def pump_x(i):
            x_loads[i].wait()
            xsend_ref[i] = xstage_ref[...].astype(jnp.bfloat16)
            if i + 1 < N_DEV - 1:
                x_loads[i + 1] = start_x((my + 2 + i) % N_DEV)
            if not _NO_COMM:
                send_block(i + 1)

        def recv_done(slot):
            return pltpu.make_async_remote_copy(
                src_ref=xg_ref.at[slot],
                dst_ref=xg_ref.at[slot],
                send_sem=send_sems.at[slot],
                recv_sem=recv_sems.at[slot],
                device_id=(my,),
                device_id_type=pl.DeviceIdType.MESH,
            )

        amax = jnp.float32(0.0)
        for step, (ki, nb) in enumerate(flat):
            slot, _ = k_seq[ki]
            if not _NO_COMM and slot is not None and nb == 0:
                recv_done(slot).wait_recv()
            if nb == 0:
                if slot is None or _NO_COMM:
                    a = xsend_ref[N_DEV - 1].astype(jnp.float32)
                else:
                    a = xg_ref[slot].astype(jnp.float32)
            if step + 1 < len(flat) and step + 1 not in w_descs:
                start_w(step + 1)
            if 0 <= step - 1 < N_DEV - 1:
                pump_x(step - 1)
            w_descs[step].wait()
            contrib = jnp.dot(a, wbuf_ref[step % NSLOT],
                              preferred_element_type=jnp.float32)
            nsl = pl.ds(nb * N_CHUNK, N_CHUNK)
            if ki == 0:
                out_ref[:, nsl] = contrib
            else:
                acc = out_ref[:, nsl] + contrib
                out_ref[:, nsl] = acc
                if ki == N_DEV - 1:
                    amax = jnp.maximum(amax, jnp.max(jnp.abs(acc)))

        for rdma in x_sends:
            rdma.wait_send()

        a_sends = []
        if _NO_COMM or _NO_EPI:
            g = amax
        else:
            amax_ref[N_DEV - 1] = jnp.full((8, 128), amax, jnp.float32)
            for dj in range(1, N_DEV):
                j = (my + dj) % N_DEV
                rdma = pltpu.make_async_remote_copy(
                    src_ref=amax_ref.at[N_DEV - 1],
                    dst_ref=amax_ref.at[dj - 1],
                    send_sem=a_send_sems.at[dj - 1],
                    recv_sem=a_recv_sems.at[dj - 1],
                    device_id=(j,),
                    device_id_type=pl.DeviceIdType.MESH,
                )
                rdma.start()
                a_sends.append(rdma)
            for s in range(N_DEV - 1):
                pltpu.make_async_remote_copy(
                    src_ref=amax_ref.at[s],
                    dst_ref=amax_ref.at[s],
                    send_sem=a_send_sems.at[s],
                    recv_sem=a_recv_sems.at[s],
                    device_id=(my,),
                    device_id_type=pl.DeviceIdType.MESH,
                ).wait_recv()
            g = jnp.max(amax_ref[...])

        if not _NO_EPI:
            scale = g / 448.0
            inv = 1.0 / scale
            for nb in range(NB):
                nsl = pl.ds(nb * N_CHUNK, N_CHUNK)
                v = out_ref[:, nsl]
                q = jnp.clip(v * inv, -448.0, 448.0).astype(jnp.float8_e4m3fn)
                out_ref[:, nsl] = q.astype(jnp.float32) * scale

        for rdma in a_sends:
            rdma.wait_send()

    return pl.pallas_call(
        body,
        out_shape=jax.ShapeDtypeStruct((M_PER, N_TOTAL), jnp.float32),
        in_specs=[
            pl.BlockSpec(memory_space=pl.ANY),
            pl.BlockSpec(memory_space=pl.ANY),
        ],
        out_specs=pl.BlockSpec(memory_space=pltpu.VMEM),
        scratch_shapes=[
            pltpu.VMEM((M_PER, K_BLK), jnp.float32),
            pltpu.VMEM((N_DEV, M_PER, K_BLK), jnp.bfloat16),
            pltpu.VMEM((N_DEV - 1, M_PER, K_BLK), jnp.bfloat16),
            pltpu.VMEM((2, K_BLK, N_CHUNK), jnp.float32),
            pltpu.VMEM((N_DEV, 8, 128), jnp.float32),
            pltpu.SemaphoreType.DMA((N_DEV - 1,)),
            pltpu.SemaphoreType.DMA((N_DEV - 1,)),
            pltpu.SemaphoreType.DMA((N_DEV - 1,)),
            pltpu.SemaphoreType.DMA((N_DEV - 1,)),
            pltpu.SemaphoreType.DMA((2,)),
            pltpu.SemaphoreType.DMA,
        ],
        compiler_params=pltpu.CompilerParams(
            vmem_limit_bytes=100 * 1024 * 1024,
            **({} if _NO_COMM else {"collective_id": 0}),
        ),
    )(x, w_mat)


# device time: 137352 ns/iter; 1.0726x vs baseline; 1.0726x over previous
import os

import jax
import jax.numpy as jnp
from jax import lax
from jax.experimental import pallas as pl
from jax.experimental.pallas import tpu as pltpu

_NO_COMM = bool(int(os.environ.get("A2A_NO_COMM", "0")))
_NO_EPI = bool(int(os.environ.get("A2A_NO_EPI", "0")))
_COMM_ONLY = bool(int(os.environ.get("A2A_COMM_ONLY", "0")))

N_DEV = 4
M_PER = 1024
K_BLK = 1024
N_TOTAL = 8192
N_CHUNK = 2048
NB = N_TOTAL // N_CHUNK


def kernel(x, w_mat):
    def body(x_ref, w_ref, out_ref, xstage_ref, xsend_ref, xg_ref, wbuf_ref,
             amax_ref, send_sems, recv_sems, a_send_sems, a_recv_sems,
             w_sems, x_sems):
        my = lax.axis_index("i")

        k_seq = [
            (None, my * K_BLK),
            (0, ((my - 1) % N_DEV) * K_BLK),
            (2, ((my + 1) % N_DEV) * K_BLK),
            (1, ((my + 2) % N_DEV) * K_BLK),
        ]
        flat = [(ki, nb) for ki in range(N_DEV) for nb in range(NB)]
        NSLOT = 2
        w_descs = {}

        def start_w(step):
            ki, nb = flat[step]
            _, krow = k_seq[ki]
            d = pltpu.make_async_copy(
                w_ref.at[pl.ds(krow, K_BLK), pl.ds(nb * N_CHUNK, N_CHUNK)],
                wbuf_ref.at[step % NSLOT],
                w_sems.at[step % NSLOT],
            )
            d.start()
            w_descs[step] = d

        def start_x(block_j):
            d = pltpu.make_async_copy(
                x_ref.at[pl.ds(block_j * M_PER, M_PER), :],
                xstage_ref,
                x_sems,
            )
            d.start()
            return d

        xdl = start_x(my)
        start_w(0)
        start_w(1)

        if not _NO_COMM:
            barrier = pltpu.get_barrier_semaphore()
            for dj in range(1, N_DEV):
                pl.semaphore_signal(
                    barrier, inc=1,
                    device_id=((my + dj) % N_DEV,),
                    device_id_type=pl.DeviceIdType.MESH,
                )
            pl.semaphore_wait(barrier, N_DEV - 1)

        x_sends = []

        def send_block(dj):
            j = (my + dj) % N_DEV
            rdma = pltpu.make_async_remote_copy(
                src_ref=xsend_ref.at[dj - 1],
                dst_ref=xg_ref.at[dj - 1],
                send_sem=send_sems.at[dj - 1],
                recv_sem=recv_sems.at[dj - 1],
                device_id=(j,),
                device_id_type=pl.DeviceIdType.MESH,
            )
            rdma.start()
            x_sends.append(rdma)

        xdl.wait()
        xsend_ref[N_DEV - 1] = xstage_ref[...].astype(jnp.bfloat16)
        x_loads = [start_x((my + 1) % N_DEV), None, None]

        def pump_x(i):
            x_loads[i].wait()
            xsend_ref[i] = xstage_ref[...].astype(jnp.bfloat16)
            if i + 1 < N_DEV - 1:
                x_loads[i + 1] = start_x((my + 2 + i) % N_DEV)
            if not _NO_COMM:
                send_block(i + 1)

        def recv_done(slot):
            return pltpu.make_async_remote_copy(
                src_ref=xg_ref.at[slot],
                dst_ref=xg_ref.at[slot],
                send_sem=send_sems.at[slot],
                recv_sem=recv_sems.at[slot],
                device_id=(my,),
                device_id_type=pl.DeviceIdType.MESH,
            )

        amax = jnp.float32(0.0)
        if _COMM_ONLY:
            for i in range(N_DEV - 1):
                pump_x(i)
            for s in range(N_DEV - 1):
                recv_done(s).wait_recv()
            out_ref[...] = jnp.zeros((M_PER, N_TOTAL), jnp.bfloat16)
            for st in (0, 1):
                w_descs[st].wait()
            flat_run = []
        else:
            flat_run = flat
        for step, (ki, nb) in enumerate(flat_run):
            slot, _ = k_seq[ki]
            if not _NO_COMM and slot is not None and nb == 0:
                recv_done(slot).wait_recv()
            if nb == 0:
                if slot is None or _NO_COMM:
                    a = xsend_ref[N_DEV - 1].astype(jnp.float32)
                else:
                    a = xg_ref[slot].astype(jnp.float32)
            if step + 1 < len(flat) and step + 1 not in w_descs:
                start_w(step + 1)
            if 0 <= step - 1 < N_DEV - 1:
                pump_x(step - 1)
            w_descs[step].wait()
            contrib = jnp.dot(a, wbuf_ref[step % NSLOT],
                              preferred_element_type=jnp.float32)
            nsl = pl.ds(nb * N_CHUNK, N_CHUNK)
            if ki == 0:
                out_ref[:, nsl] = contrib.astype(jnp.bfloat16)
            else:
                acc = out_ref[:, nsl] + contrib
                out_ref[:, nsl] = acc.astype(jnp.bfloat16)
                if ki == N_DEV - 1:
                    amax = jnp.maximum(amax, jnp.max(jnp.abs(acc)))

        for rdma in x_sends:
            rdma.wait_send()

        a_sends = []
        if _NO_COMM or _NO_EPI:
            g = amax
        else:
            amax_ref[N_DEV - 1] = jnp.full((8, 128), amax, jnp.float32)
            for dj in range(1, N_DEV):
                j = (my + dj) % N_DEV
                rdma = pltpu.make_async_remote_copy(
                    src_ref=amax_ref.at[N_DEV - 1],
                    dst_ref=amax_ref.at[dj - 1],
                    send_sem=a_send_sems.at[dj - 1],
                    recv_sem=a_recv_sems.at[dj - 1],
                    device_id=(j,),
                    device_id_type=pl.DeviceIdType.MESH,
                )
                rdma.start()
                a_sends.append(rdma)
            for s in range(N_DEV - 1):
                pltpu.make_async_remote_copy(
                    src_ref=amax_ref.at[s],
                    dst_ref=amax_ref.at[s],
                    send_sem=a_send_sems.at[s],
                    recv_sem=a_recv_sems.at[s],
                    device_id=(my,),
                    device_id_type=pl.DeviceIdType.MESH,
                ).wait_recv()
            g = jnp.max(amax_ref[...])

        if not _NO_EPI and not _COMM_ONLY:
            scale = g / 448.0
            inv = 1.0 / scale
            for nb in range(NB):
                nsl = pl.ds(nb * N_CHUNK, N_CHUNK)
                v = out_ref[:, nsl].astype(jnp.float32)
                q = jnp.clip(v * inv, -448.0, 448.0).astype(jnp.float8_e4m3fn)
                out_ref[:, nsl] = (q.astype(jnp.float32) * scale).astype(jnp.bfloat16)

        for rdma in a_sends:
            rdma.wait_send()

    return pl.pallas_call(
        body,
        out_shape=jax.ShapeDtypeStruct((M_PER, N_TOTAL), jnp.bfloat16),
        in_specs=[
            pl.BlockSpec(memory_space=pl.ANY),
            pl.BlockSpec(memory_space=pl.ANY),
        ],
        out_specs=pl.BlockSpec(memory_space=pltpu.VMEM),
        scratch_shapes=[
            pltpu.VMEM((M_PER, K_BLK), jnp.float32),
            pltpu.VMEM((N_DEV, M_PER, K_BLK), jnp.bfloat16),
            pltpu.VMEM((N_DEV - 1, M_PER, K_BLK), jnp.bfloat16),
            pltpu.VMEM((2, K_BLK, N_CHUNK), jnp.float32),
            pltpu.VMEM((N_DEV, 8, 128), jnp.float32),
            pltpu.SemaphoreType.DMA((N_DEV - 1,)),
            pltpu.SemaphoreType.DMA((N_DEV - 1,)),
            pltpu.SemaphoreType.DMA((N_DEV - 1,)),
            pltpu.SemaphoreType.DMA((N_DEV - 1,)),
            pltpu.SemaphoreType.DMA((2,)),
            pltpu.SemaphoreType.DMA,
        ],
        compiler_params=pltpu.CompilerParams(
            vmem_limit_bytes=100 * 1024 * 1024,
            **({} if _NO_COMM else {"collective_id": 0}),
        ),
    )(x, w_mat)
